# Initial kernel scaffold; baseline (speedup 1.0000x reference)
import jax
import jax.numpy as jnp
from jax.experimental import pallas as pl


def kernel(x, pos, batch):
    raise NotImplementedError("write your pallas kernel here")


# speedup vs baseline: 15.2297x; 15.2297x over previous
import functools

import jax
import jax.numpy as jnp
from jax import lax
from jax.experimental import pallas as pl
from jax.experimental.pallas import tpu as pltpu

_RATIO = 0.5
_R = 1.0
_K = 64


def _fps_body(pts_ref, idx_ref, dists_ref, *, n_samples, n_valid, s_pad):
    rw = pts_ref.shape[1]
    row_i = lax.broadcasted_iota(jnp.int32, (rw, 128), 0)
    col_i = lax.broadcasted_iota(jnp.int32, (rw, 128), 1)
    flat = row_i * 128 + col_i
    valid = flat < n_valid
    dists_ref[...] = jnp.where(valid, jnp.inf, -jnp.inf).astype(jnp.float32)

    def zbody(i, _):
        idx_ref[i] = jnp.int32(0)
        return 0

    lax.fori_loop(n_samples - 1, s_pad, zbody, 0)
    idx_ref[0] = jnp.int32(0)

    def body(i, last):
        sel = flat == last
        p0 = pts_ref[0]
        p1 = pts_ref[1]
        p2 = pts_ref[2]
        p3 = pts_ref[3]
        p4 = pts_ref[4]
        p5 = pts_ref[5]
        c0 = jnp.sum(jnp.where(sel, p0, 0.0))
        c1 = jnp.sum(jnp.where(sel, p1, 0.0))
        c2 = jnp.sum(jnp.where(sel, p2, 0.0))
        c3 = jnp.sum(jnp.where(sel, p3, 0.0))
        c4 = jnp.sum(jnp.where(sel, p4, 0.0))
        c5 = jnp.sum(jnp.where(sel, p5, 0.0))
        d0 = p0 - c0
        d1 = p1 - c1
        d2 = p2 - c2
        d3 = p3 - c3
        d4 = p4 - c4
        d5 = p5 - c5
        s0 = d0 * d0
        s1 = d1 * d1
        s2 = d2 * d2
        s3 = d3 * d3
        s4 = d4 * d4
        s5 = d5 * d5
        d = ((s0 + s4) + s2) + ((s1 + s5) + s3)
        dn = jnp.minimum(dists_ref[...], d)
        dists_ref[...] = dn
        m = jnp.max(dn)
        nxt = jnp.min(jnp.where(dn == m, flat, jnp.int32(2**30)))
        idx_ref[i] = nxt
        return nxt

    lax.fori_loop(1, n_samples, body, jnp.int32(0))


def _fps(planes, n_samples, n_valid, s_pad):
    rw = planes.shape[1]
    return pl.pallas_call(
        functools.partial(
            _fps_body, n_samples=n_samples, n_valid=n_valid, s_pad=s_pad
        ),
        out_shape=jax.ShapeDtypeStruct((s_pad,), jnp.int32),
        out_specs=pl.BlockSpec(memory_space=pltpu.SMEM),
        scratch_shapes=[pltpu.VMEM((rw, 128), jnp.float32)],
    )(planes)


def _radius_mean_body(centers_ref, ptsT_ref, xaug_ref, out_ref, thr_ref, *, n_valid, k):
    cb = centers_ref[...]
    pts = ptsT_ref[...]
    c2 = jnp.sum(cb * cb, axis=1, keepdims=True)
    p2 = jnp.sum(pts * pts, axis=0, keepdims=True)
    cp = lax.dot_general(
        cb, pts, (((1,), (0,)), ((), ())), preferred_element_type=jnp.float32
    )
    d2 = (c2 + p2) - 2.0 * cp
    col = lax.broadcasted_iota(jnp.int32, d2.shape, 1)
    mask = (d2 <= _R * _R) & (col < n_valid)
    cntf = jnp.sum(jnp.where(mask, 1.0, 0.0), axis=1, keepdims=True)
    kf = jnp.float32(k)
    maxcnt = jnp.max(cntf)

    shifted = jnp.where(mask, d2 + 2.0, jnp.inf)
    bits = lax.bitcast_convert_type(shifted, jnp.int32)
    infb = jnp.int32(0x7F800000)

    thr_ref[...] = jnp.full(thr_ref.shape, infb, jnp.int32)

    @pl.when(maxcnt > kf)
    def _():
        lo0 = jnp.full(thr_ref.shape, 0x3F800000, jnp.int32)
        hi0 = jnp.full(thr_ref.shape, 0x40400000, jnp.int32)

        def bbody(_, lh):
            lo, hi = lh
            mid = lo + (hi - lo) // 2
            c = jnp.sum(jnp.where(bits <= mid, 1.0, 0.0), axis=1, keepdims=True)
            ge = c >= kf
            return jnp.where(ge, lo, mid + 1), jnp.where(ge, mid, hi)

        lo, _ = lax.fori_loop(0, 24, bbody, (lo0, hi0))
        thr_ref[...] = jnp.where(cntf > kf, lo, infb)

    include = jnp.where(mask & (bits <= thr_ref[...]), 1.0, 0.0)
    sums = lax.dot_general(
        include, xaug_ref[...], (((1,), (0,)), ((), ())),
        preferred_element_type=jnp.float32,
    )
    cnt_inc = sums[:, 3:4]
    out_ref[...] = sums / jnp.maximum(cnt_inc, 1.0)


def _radius_mean(centers8, ptsT, xaug, n_valid, bc):
    s_pad = centers8.shape[0]
    np_ = ptsT.shape[1]
    grid = (s_pad // bc,)
    return pl.pallas_call(
        functools.partial(_radius_mean_body, n_valid=n_valid, k=_K),
        grid=grid,
        in_specs=[
            pl.BlockSpec((bc, 8), lambda i: (i, 0)),
            pl.BlockSpec((8, np_), lambda i: (0, 0)),
            pl.BlockSpec((np_, 8), lambda i: (0, 0)),
        ],
        out_specs=pl.BlockSpec((bc, 8), lambda i: (i, 0)),
        out_shape=jax.ShapeDtypeStruct((s_pad, 8), jnp.float32),
        scratch_shapes=[pltpu.VMEM((bc, 1), jnp.int32)],
    )(centers8, ptsT, xaug)


def kernel(x, pos, batch):
    n = x.shape[0]
    s = int(_RATIO * n)
    np_ = ((n + 1023) // 1024) * 1024
    rw = np_ // 128
    s_pad = ((s + 255) // 256) * 256
    bc = 256

    pos6d = jnp.concatenate([x, pos], axis=1)
    pts_t = jnp.pad(pos6d, ((0, np_ - n), (0, 0))).T
    planes = pts_t.reshape(6, rw, 128)

    idx = _fps(planes, s, n, s_pad)

    centers6 = jnp.take(pos6d, idx, axis=0)
    centers8 = jnp.pad(centers6, ((0, 0), (0, 2)))
    ptsT8 = jnp.pad(pts_t, ((0, 2), (0, 0)))
    xaug = jnp.pad(
        jnp.concatenate([x, jnp.ones((n, 1), jnp.float32)], axis=1),
        ((0, np_ - n), (0, 4)),
    )

    out8 = _radius_mean(centers8, ptsT8, xaug, n, bc)

    x_centers = out8[:s, :3]
    pos_out = jnp.take(pos, idx[:s], axis=0)
    batch_out = jnp.zeros((s,), batch.dtype)
    return (x_centers, pos_out, batch_out)


# speedup vs baseline: 23.0248x; 1.5118x over previous
import functools

import jax
import jax.numpy as jnp
from jax import lax
from jax.experimental import pallas as pl
from jax.experimental.pallas import tpu as pltpu
from jax.experimental.pallas import tpu_sc as plsc

_RATIO = 0.5
_R = 1.0
_K = 64


def _fps_body(pts_ref, rows_ref, idx_ref, *, n_samples, n_valid, s_pad):
    rw = pts_ref.shape[1]
    row_i = lax.broadcasted_iota(jnp.int32, (rw, 128), 0)
    col_i = lax.broadcasted_iota(jnp.int32, (rw, 128), 1)
    flat = row_i * 128 + col_i
    valid = flat < n_valid
    dists0 = jnp.where(valid, jnp.inf, -jnp.inf).astype(jnp.float32)

    def zbody(i, _):
        idx_ref[i] = jnp.int32(0)
        return 0

    lax.fori_loop(n_samples, s_pad, zbody, 0)
    idx_ref[0] = jnp.int32(0)

    def body(i, carry):
        last, dists = carry
        base = last * 6
        c0 = rows_ref[base]
        c1 = rows_ref[base + 1]
        c2 = rows_ref[base + 2]
        c3 = rows_ref[base + 3]
        c4 = rows_ref[base + 4]
        c5 = rows_ref[base + 5]
        d0 = pts_ref[0] - c0
        d1 = pts_ref[1] - c1
        d2 = pts_ref[2] - c2
        d3 = pts_ref[3] - c3
        d4 = pts_ref[4] - c4
        d5 = pts_ref[5] - c5
        s0 = d0 * d0
        s1 = d1 * d1
        s2 = d2 * d2
        s3 = d3 * d3
        s4 = d4 * d4
        s5 = d5 * d5
        d = ((s0 + s4) + s2) + ((s1 + s5) + s3)
        dn = jnp.minimum(dists, d)
        m_col = jnp.max(dn, axis=0, keepdims=True)
        r_col = jnp.min(
            jnp.where(dn == m_col, row_i, jnp.int32(2**30)), axis=0, keepdims=True
        )
        flat_col = r_col * 128 + col_i[0:1, :]
        m = jnp.max(m_col)
        nxt = jnp.min(jnp.where(m_col == m, flat_col, jnp.int32(2**30)))
        idx_ref[i] = nxt
        return (nxt, dn)

    lax.fori_loop(1, n_samples, body, (jnp.int32(0), dists0))


def _fps(planes, rows, n_samples, n_valid, s_pad):
    rw = planes.shape[1]
    return pl.pallas_call(
        functools.partial(
            _fps_body, n_samples=n_samples, n_valid=n_valid, s_pad=s_pad
        ),
        out_shape=jax.ShapeDtypeStruct((s_pad,), jnp.int32),
        in_specs=[
            pl.BlockSpec(memory_space=pltpu.VMEM),
            pl.BlockSpec(memory_space=pltpu.SMEM),
        ],
        out_specs=pl.BlockSpec(memory_space=pltpu.SMEM),
    )(planes, rows)


def _sc_gather(table, idx):
    info = plsc.get_sparse_core_info()
    nc, ns = info.num_cores, info.num_subcores
    nw = nc * ns
    b = idx.shape[0]
    bpw = b // nw
    d = table.shape[1]
    mesh = plsc.VectorSubcoreMesh(core_axis_name="c", subcore_axis_name="s")

    @functools.partial(
        pl.kernel,
        mesh=mesh,
        out_type=jax.ShapeDtypeStruct((b, d), jnp.float32),
        scratch_types=[
            pltpu.VMEM((bpw,), jnp.int32),
            pltpu.VMEM((bpw, d), jnp.float32),
            pltpu.SemaphoreType.DMA,
        ],
    )
    def k(table_hbm, idx_hbm, out_hbm, idx_v, rows_v, sem):
        wid = lax.axis_index("s") * nc + lax.axis_index("c")
        base = wid * bpw
        pltpu.sync_copy(idx_hbm.at[pl.ds(base, bpw)], idx_v)
        pltpu.async_copy(table_hbm.at[idx_v], rows_v, sem).wait()
        pltpu.sync_copy(rows_v, out_hbm.at[pl.ds(base, bpw)])

    return k(table, idx)


def _radius_mean_body(centers_ref, ptsT_ref, xaug_ref, out_ref, thr_ref, *, n_valid, k):
    cb = centers_ref[...]
    pts = ptsT_ref[...]
    c2 = jnp.sum(cb * cb, axis=1, keepdims=True)
    p2 = jnp.sum(pts * pts, axis=0, keepdims=True)
    cp = lax.dot_general(
        cb, pts, (((1,), (0,)), ((), ())), preferred_element_type=jnp.float32
    )
    d2 = (c2 + p2) - 2.0 * cp
    col = lax.broadcasted_iota(jnp.int32, d2.shape, 1)
    mask = (d2 <= _R * _R) & (col < n_valid)
    cntf = jnp.sum(jnp.where(mask, 1.0, 0.0), axis=1, keepdims=True)
    kf = jnp.float32(k)
    maxcnt = jnp.max(cntf)

    shifted = jnp.where(mask, d2 + 2.0, jnp.inf)
    bits = lax.bitcast_convert_type(shifted, jnp.int32)
    infb = jnp.int32(0x7F800000)

    thr_ref[...] = jnp.full(thr_ref.shape, infb, jnp.int32)

    @pl.when(maxcnt > kf)
    def _():
        lo0 = jnp.full(thr_ref.shape, 0x3F800000, jnp.int32)
        hi0 = jnp.full(thr_ref.shape, 0x40400000, jnp.int32)

        def bbody(_, lh):
            lo, hi = lh
            mid = lo + (hi - lo) // 2
            c = jnp.sum(jnp.where(bits <= mid, 1.0, 0.0), axis=1, keepdims=True)
            ge = c >= kf
            return jnp.where(ge, lo, mid + 1), jnp.where(ge, mid, hi)

        lo, _ = lax.fori_loop(0, 24, bbody, (lo0, hi0))
        thr_ref[...] = jnp.where(cntf > kf, lo, infb)

    include = jnp.where(mask & (bits <= thr_ref[...]), 1.0, 0.0)
    sums = lax.dot_general(
        include, xaug_ref[...], (((1,), (0,)), ((), ())),
        preferred_element_type=jnp.float32,
    )
    cnt_inc = sums[:, 3:4]
    out_ref[...] = sums / jnp.maximum(cnt_inc, 1.0)


def _radius_mean(centers8, ptsT, xaug, n_valid, bc):
    s_pad = centers8.shape[0]
    np_ = ptsT.shape[1]
    grid = (s_pad // bc,)
    return pl.pallas_call(
        functools.partial(_radius_mean_body, n_valid=n_valid, k=_K),
        grid=grid,
        in_specs=[
            pl.BlockSpec((bc, 8), lambda i: (i, 0)),
            pl.BlockSpec((8, np_), lambda i: (0, 0)),
            pl.BlockSpec((np_, 8), lambda i: (0, 0)),
        ],
        out_specs=pl.BlockSpec((bc, 8), lambda i: (i, 0)),
        out_shape=jax.ShapeDtypeStruct((s_pad, 8), jnp.float32),
        scratch_shapes=[pltpu.VMEM((bc, 1), jnp.int32)],
    )(centers8, ptsT, xaug)


def kernel(x, pos, batch):
    n = x.shape[0]
    s = int(_RATIO * n)
    np_ = ((n + 1023) // 1024) * 1024
    rw = np_ // 128
    s_pad = ((s + 255) // 256) * 256
    bc = 256

    pos6d = jnp.concatenate([x, pos], axis=1)
    pts_t = jnp.pad(pos6d, ((0, np_ - n), (0, 0))).T
    planes = pts_t.reshape(6, rw, 128)
    rows6 = jnp.pad(pos6d, ((0, np_ - n), (0, 0))).reshape(-1)

    idx = _fps(planes, rows6, s, n, s_pad)

    table = jnp.pad(
        jnp.concatenate([pos6d, pos], axis=1), ((0, np_ - n), (0, 119))
    )
    gathered = _sc_gather(table, idx)
    centers8 = jnp.concatenate(
        [gathered[:, :6], jnp.zeros((s_pad, 2), jnp.float32)], axis=1
    )
    ptsT8 = jnp.pad(pts_t, ((0, 2), (0, 0)))
    xaug = jnp.pad(
        jnp.concatenate([x, jnp.ones((n, 1), jnp.float32)], axis=1),
        ((0, np_ - n), (0, 4)),
    )

    out8 = _radius_mean(centers8, ptsT8, xaug, n, bc)

    x_centers = out8[:s, :3]
    pos_out = gathered[:s, 6:9]
    batch_out = jnp.zeros((s,), batch.dtype)
    return (x_centers, pos_out, batch_out)


# speedup vs baseline: 23.0720x; 1.0021x over previous
import functools

import jax
import jax.numpy as jnp
from jax import lax
from jax.experimental import pallas as pl
from jax.experimental.pallas import tpu as pltpu
from jax.experimental.pallas import tpu_sc as plsc

_RATIO = 0.5
_R = 1.0
_K = 64


def _fps_body(pts_ref, rows_ref, idx_ref, *, n_samples, n_valid, s_pad):
    rw = pts_ref.shape[1]
    row_i = lax.broadcasted_iota(jnp.int32, (rw, 128), 0)
    col_i = lax.broadcasted_iota(jnp.int32, (rw, 128), 1)
    flat = row_i * 128 + col_i
    valid = flat < n_valid
    row_f = row_i.astype(jnp.float32)
    lane_f = col_i[0:1, :].astype(jnp.float32)
    dists0 = jnp.where(valid, jnp.inf, -jnp.inf).astype(jnp.float32)

    def zbody(i, _):
        idx_ref[i] = jnp.int32(0)
        return 0

    lax.fori_loop(n_samples, s_pad, zbody, 0)
    idx_ref[0] = jnp.int32(0)

    def body(i, carry):
        last, dists = carry
        base = last * 6
        c0 = rows_ref[base]
        c1 = rows_ref[base + 1]
        c2 = rows_ref[base + 2]
        c3 = rows_ref[base + 3]
        c4 = rows_ref[base + 4]
        c5 = rows_ref[base + 5]
        d0 = pts_ref[0] - c0
        d1 = pts_ref[1] - c1
        d2 = pts_ref[2] - c2
        d3 = pts_ref[3] - c3
        d4 = pts_ref[4] - c4
        d5 = pts_ref[5] - c5
        s0 = d0 * d0
        s1 = d1 * d1
        s2 = d2 * d2
        s3 = d3 * d3
        s4 = d4 * d4
        s5 = d5 * d5
        d = ((s0 + s4) + s2) + ((s1 + s5) + s3)
        dn = jnp.minimum(dists, d)
        m_col = jnp.max(dn, axis=0, keepdims=True)
        r_col = jnp.min(
            jnp.where(dn == m_col, row_f, jnp.float32(1e9)), axis=0, keepdims=True
        )
        flat_col = r_col * 128.0 + lane_f
        m = jnp.max(m_col)
        nxt_f = jnp.min(jnp.where(m_col == m, flat_col, jnp.float32(1e9)))
        nxt = nxt_f.astype(jnp.int32)
        idx_ref[i] = nxt
        return (nxt, dn)

    lax.fori_loop(1, n_samples, body, (jnp.int32(0), dists0))


def _fps(planes, rows, n_samples, n_valid, s_pad):
    rw = planes.shape[1]
    return pl.pallas_call(
        functools.partial(
            _fps_body, n_samples=n_samples, n_valid=n_valid, s_pad=s_pad
        ),
        out_shape=jax.ShapeDtypeStruct((s_pad,), jnp.int32),
        in_specs=[
            pl.BlockSpec(memory_space=pltpu.VMEM),
            pl.BlockSpec(memory_space=pltpu.SMEM),
        ],
        out_specs=pl.BlockSpec(memory_space=pltpu.SMEM),
    )(planes, rows)


def _sc_gather(table, idx):
    info = plsc.get_sparse_core_info()
    nc, ns = info.num_cores, info.num_subcores
    nw = nc * ns
    b = idx.shape[0]
    bpw = b // nw
    d = table.shape[1]
    mesh = plsc.VectorSubcoreMesh(core_axis_name="c", subcore_axis_name="s")

    @functools.partial(
        pl.kernel,
        mesh=mesh,
        out_type=jax.ShapeDtypeStruct((b, d), jnp.float32),
        scratch_types=[
            pltpu.VMEM((bpw,), jnp.int32),
            pltpu.VMEM((bpw, d), jnp.float32),
            pltpu.SemaphoreType.DMA,
        ],
    )
    def k(table_hbm, idx_hbm, out_hbm, idx_v, rows_v, sem):
        wid = lax.axis_index("s") * nc + lax.axis_index("c")
        base = wid * bpw
        pltpu.sync_copy(idx_hbm.at[pl.ds(base, bpw)], idx_v)
        pltpu.async_copy(table_hbm.at[idx_v], rows_v, sem).wait()
        pltpu.sync_copy(rows_v, out_hbm.at[pl.ds(base, bpw)])

    return k(table, idx)


def _radius_mean_body(centers_ref, ptsT_ref, xaug_ref, out_ref, thr_ref, *, n_valid, k):
    cb = centers_ref[...]
    pts = ptsT_ref[...]
    c2 = jnp.sum(cb * cb, axis=1, keepdims=True)
    p2 = jnp.sum(pts * pts, axis=0, keepdims=True)
    cp = lax.dot_general(
        cb, pts, (((1,), (0,)), ((), ())), preferred_element_type=jnp.float32
    )
    d2 = (c2 + p2) - 2.0 * cp
    col = lax.broadcasted_iota(jnp.int32, d2.shape, 1)
    mask = (d2 <= _R * _R) & (col < n_valid)
    cntf = jnp.sum(jnp.where(mask, 1.0, 0.0), axis=1, keepdims=True)
    kf = jnp.float32(k)
    maxcnt = jnp.max(cntf)

    shifted = jnp.where(mask, d2 + 2.0, jnp.inf)
    bits = lax.bitcast_convert_type(shifted, jnp.int32)
    infb = jnp.int32(0x7F800000)

    thr_ref[...] = jnp.full(thr_ref.shape, infb, jnp.int32)

    @pl.when(maxcnt > kf)
    def _():
        lo0 = jnp.full(thr_ref.shape, 0x3F800000, jnp.int32)
        hi0 = jnp.full(thr_ref.shape, 0x40400000, jnp.int32)

        def bbody(_, lh):
            lo, hi = lh
            mid = lo + (hi - lo) // 2
            c = jnp.sum(jnp.where(bits <= mid, 1.0, 0.0), axis=1, keepdims=True)
            ge = c >= kf
            return jnp.where(ge, lo, mid + 1), jnp.where(ge, mid, hi)

        lo, _ = lax.fori_loop(0, 24, bbody, (lo0, hi0))
        thr_ref[...] = jnp.where(cntf > kf, lo, infb)

    include = jnp.where(mask & (bits <= thr_ref[...]), 1.0, 0.0)
    sums = lax.dot_general(
        include, xaug_ref[...], (((1,), (0,)), ((), ())),
        preferred_element_type=jnp.float32,
    )
    cnt_inc = sums[:, 3:4]
    out_ref[...] = sums / jnp.maximum(cnt_inc, 1.0)


def _radius_mean(centers8, ptsT, xaug, n_valid, bc):
    s_pad = centers8.shape[0]
    np_ = ptsT.shape[1]
    grid = (s_pad // bc,)
    return pl.pallas_call(
        functools.partial(_radius_mean_body, n_valid=n_valid, k=_K),
        grid=grid,
        in_specs=[
            pl.BlockSpec((bc, 8), lambda i: (i, 0)),
            pl.BlockSpec((8, np_), lambda i: (0, 0)),
            pl.BlockSpec((np_, 8), lambda i: (0, 0)),
        ],
        out_specs=pl.BlockSpec((bc, 8), lambda i: (i, 0)),
        out_shape=jax.ShapeDtypeStruct((s_pad, 8), jnp.float32),
        scratch_shapes=[pltpu.VMEM((bc, 1), jnp.int32)],
    )(centers8, ptsT, xaug)


def kernel(x, pos, batch):
    n = x.shape[0]
    s = int(_RATIO * n)
    np_ = ((n + 1023) // 1024) * 1024
    rw = np_ // 128
    s_pad = ((s + 255) // 256) * 256
    bc = 256

    pos6d = jnp.concatenate([x, pos], axis=1)
    pts_t = jnp.pad(pos6d, ((0, np_ - n), (0, 0))).T
    planes = pts_t.reshape(6, rw, 128)
    rows6 = jnp.pad(pos6d, ((0, np_ - n), (0, 0))).reshape(-1)

    idx = _fps(planes, rows6, s, n, s_pad)

    table = jnp.pad(
        jnp.concatenate([pos6d, pos], axis=1), ((0, np_ - n), (0, 119))
    )
    gathered = _sc_gather(table, idx)
    centers8 = jnp.concatenate(
        [gathered[:, :6], jnp.zeros((s_pad, 2), jnp.float32)], axis=1
    )
    ptsT8 = jnp.pad(pts_t, ((0, 2), (0, 0)))
    xaug = jnp.pad(
        jnp.concatenate([x, jnp.ones((n, 1), jnp.float32)], axis=1),
        ((0, np_ - n), (0, 4)),
    )

    out8 = _radius_mean(centers8, ptsT8, xaug, n, bc)

    x_centers = out8[:s, :3]
    pos_out = gathered[:s, 6:9]
    batch_out = jnp.zeros((s,), batch.dtype)
    return (x_centers, pos_out, batch_out)


# speedup vs baseline: 25.1777x; 1.0913x over previous
import functools

import jax
import jax.numpy as jnp
from jax import lax
from jax.experimental import pallas as pl
from jax.experimental.pallas import tpu as pltpu
from jax.experimental.pallas import tpu_sc as plsc

_RATIO = 0.5
_R = 1.0
_K = 64


def _fps_body(pts_ref, rows_ref, idx_ref, *, n_samples, n_valid, s_pad):
    rw = pts_ref.shape[1]
    row_i = lax.broadcasted_iota(jnp.int32, (rw, 128), 0)
    col_i = lax.broadcasted_iota(jnp.int32, (rw, 128), 1)
    flat = row_i * 128 + col_i
    valid = flat < n_valid
    row_f = row_i.astype(jnp.float32)
    lane_f = col_i[0:1, :].astype(jnp.float32)
    dists0 = jnp.where(valid, jnp.inf, -jnp.inf).astype(jnp.float32)

    def zbody(i, _):
        idx_ref[i] = jnp.int32(0)
        return 0

    lax.fori_loop(n_samples, s_pad, zbody, 0)
    idx_ref[0] = jnp.int32(0)

    def body(i, carry):
        last, dists = carry
        base = last * 6
        c0 = rows_ref[base]
        c1 = rows_ref[base + 1]
        c2 = rows_ref[base + 2]
        c3 = rows_ref[base + 3]
        c4 = rows_ref[base + 4]
        c5 = rows_ref[base + 5]
        d0 = pts_ref[0] - c0
        d1 = pts_ref[1] - c1
        d2 = pts_ref[2] - c2
        d3 = pts_ref[3] - c3
        d4 = pts_ref[4] - c4
        d5 = pts_ref[5] - c5
        s0 = d0 * d0
        s1 = d1 * d1
        s2 = d2 * d2
        s3 = d3 * d3
        s4 = d4 * d4
        s5 = d5 * d5
        d = ((s0 + s4) + s2) + ((s1 + s5) + s3)
        dn = jnp.minimum(dists, d)
        m_col = jnp.max(dn, axis=0, keepdims=True)
        r_col = jnp.min(
            jnp.where(dn == m_col, row_f, jnp.float32(1e9)), axis=0, keepdims=True
        )
        flat_col = r_col * 128.0 + lane_f
        m = jnp.max(m_col, axis=1, keepdims=True)
        nxt_f = jnp.min(jnp.where(m_col == m, flat_col, jnp.float32(1e9)))
        nxt = nxt_f.astype(jnp.int32)
        idx_ref[i] = nxt
        return (nxt, dn)

    lax.fori_loop(1, n_samples, body, (jnp.int32(0), dists0))


def _fps(planes, rows, n_samples, n_valid, s_pad):
    rw = planes.shape[1]
    return pl.pallas_call(
        functools.partial(
            _fps_body, n_samples=n_samples, n_valid=n_valid, s_pad=s_pad
        ),
        out_shape=jax.ShapeDtypeStruct((s_pad,), jnp.int32),
        in_specs=[
            pl.BlockSpec(memory_space=pltpu.VMEM),
            pl.BlockSpec(memory_space=pltpu.SMEM),
        ],
        out_specs=pl.BlockSpec(memory_space=pltpu.SMEM),
    )(planes, rows)


def _sc_gather(table, idx):
    info = plsc.get_sparse_core_info()
    nc, ns = info.num_cores, info.num_subcores
    nw = nc * ns
    b = idx.shape[0]
    bpw = b // nw
    d = table.shape[1]
    mesh = plsc.VectorSubcoreMesh(core_axis_name="c", subcore_axis_name="s")

    @functools.partial(
        pl.kernel,
        mesh=mesh,
        out_type=jax.ShapeDtypeStruct((b, d), jnp.float32),
        scratch_types=[
            pltpu.VMEM((bpw,), jnp.int32),
            pltpu.VMEM((bpw, d), jnp.float32),
            pltpu.SemaphoreType.DMA,
        ],
    )
    def k(table_hbm, idx_hbm, out_hbm, idx_v, rows_v, sem):
        wid = lax.axis_index("s") * nc + lax.axis_index("c")
        base = wid * bpw
        pltpu.sync_copy(idx_hbm.at[pl.ds(base, bpw)], idx_v)
        pltpu.async_copy(table_hbm.at[idx_v], rows_v, sem).wait()
        pltpu.sync_copy(rows_v, out_hbm.at[pl.ds(base, bpw)])

    return k(table, idx)


def _radius_mean_body(centers_ref, ptsT_ref, xaug_ref, out_ref, thr_ref, *, n_valid, k):
    cb = centers_ref[...]
    pts = ptsT_ref[...]
    c2 = jnp.sum(cb * cb, axis=1, keepdims=True)
    p2 = jnp.sum(pts * pts, axis=0, keepdims=True)
    cp = lax.dot_general(
        cb, pts, (((1,), (0,)), ((), ())), preferred_element_type=jnp.float32
    )
    d2 = (c2 + p2) - 2.0 * cp
    col = lax.broadcasted_iota(jnp.int32, d2.shape, 1)
    mask = (d2 <= _R * _R) & (col < n_valid)
    cntf = jnp.sum(jnp.where(mask, 1.0, 0.0), axis=1, keepdims=True)
    kf = jnp.float32(k)
    maxcnt = jnp.max(cntf)

    shifted = jnp.where(mask, d2 + 2.0, jnp.inf)
    bits = lax.bitcast_convert_type(shifted, jnp.int32)
    infb = jnp.int32(0x7F800000)

    thr_ref[...] = jnp.full(thr_ref.shape, infb, jnp.int32)

    @pl.when(maxcnt > kf)
    def _():
        lo0 = jnp.full(thr_ref.shape, 0x3F800000, jnp.int32)
        hi0 = jnp.full(thr_ref.shape, 0x40400000, jnp.int32)

        def bbody(_, lh):
            lo, hi = lh
            mid = lo + (hi - lo) // 2
            c = jnp.sum(jnp.where(bits <= mid, 1.0, 0.0), axis=1, keepdims=True)
            ge = c >= kf
            return jnp.where(ge, lo, mid + 1), jnp.where(ge, mid, hi)

        lo, _ = lax.fori_loop(0, 24, bbody, (lo0, hi0))
        thr_ref[...] = jnp.where(cntf > kf, lo, infb)

    include = jnp.where(mask & (bits <= thr_ref[...]), 1.0, 0.0)
    sums = lax.dot_general(
        include, xaug_ref[...], (((1,), (0,)), ((), ())),
        preferred_element_type=jnp.float32,
    )
    cnt_inc = sums[:, 3:4]
    out_ref[...] = sums / jnp.maximum(cnt_inc, 1.0)


def _radius_mean(centers8, ptsT, xaug, n_valid, bc):
    s_pad = centers8.shape[0]
    np_ = ptsT.shape[1]
    grid = (s_pad // bc,)
    return pl.pallas_call(
        functools.partial(_radius_mean_body, n_valid=n_valid, k=_K),
        grid=grid,
        in_specs=[
            pl.BlockSpec((bc, 8), lambda i: (i, 0)),
            pl.BlockSpec((8, np_), lambda i: (0, 0)),
            pl.BlockSpec((np_, 8), lambda i: (0, 0)),
        ],
        out_specs=pl.BlockSpec((bc, 8), lambda i: (i, 0)),
        out_shape=jax.ShapeDtypeStruct((s_pad, 8), jnp.float32),
        scratch_shapes=[pltpu.VMEM((bc, 1), jnp.int32)],
    )(centers8, ptsT, xaug)


def kernel(x, pos, batch):
    n = x.shape[0]
    s = int(_RATIO * n)
    np_ = ((n + 1023) // 1024) * 1024
    rw = np_ // 128
    s_pad = ((s + 255) // 256) * 256
    bc = 256

    pos6d = jnp.concatenate([x, pos], axis=1)
    pts_t = jnp.pad(pos6d, ((0, np_ - n), (0, 0))).T
    planes = pts_t.reshape(6, rw, 128)
    rows6 = jnp.pad(pos6d, ((0, np_ - n), (0, 0))).reshape(-1)

    idx = _fps(planes, rows6, s, n, s_pad)

    table = jnp.pad(
        jnp.concatenate([pos6d, pos], axis=1), ((0, np_ - n), (0, 119))
    )
    gathered = _sc_gather(table, idx)
    centers8 = jnp.concatenate(
        [gathered[:, :6], jnp.zeros((s_pad, 2), jnp.float32)], axis=1
    )
    ptsT8 = jnp.pad(pts_t, ((0, 2), (0, 0)))
    xaug = jnp.pad(
        jnp.concatenate([x, jnp.ones((n, 1), jnp.float32)], axis=1),
        ((0, np_ - n), (0, 4)),
    )

    out8 = _radius_mean(centers8, ptsT8, xaug, n, bc)

    x_centers = out8[:s, :3]
    pos_out = gathered[:s, 6:9]
    batch_out = jnp.zeros((s,), batch.dtype)
    return (x_centers, pos_out, batch_out)
